# baseline (speedup 1.0000x reference)
import jax
import jax.numpy as jnp
from jax.experimental import pallas as pl
from jax.experimental.pallas import tpu as pltpu

_N = 8388608
_ROWS = 65536
_HEAD = 8
_NBULK = 8
_BULK_ROWS = (_ROWS - _HEAD) // _NBULK
_BULK_REM = (_ROWS - _HEAD) - _NBULK * _BULK_ROWS


def _body(t_ref, w_ref, o_ref, vbuf, hsem, *bsems):
    bulk = []
    for i in range(_NBULK):
        start = _HEAD + i * _BULK_ROWS
        n = _BULK_ROWS + (_BULK_REM if i == _NBULK - 1 else 0)
        cp = pltpu.make_async_copy(
            w_ref.at[pl.ds(start, n), :], o_ref.at[pl.ds(start, n), :], bsems[i])
        cp.start()
        bulk.append(cp)

    head_in = pltpu.make_async_copy(w_ref.at[pl.ds(0, _HEAD), :], vbuf, hsem)
    head_in.start()
    head_in.wait()

    t = t_ref[0]
    tv = jnp.full((_HEAD, 128), t, dtype=jnp.float32)
    cond = (t > 500.0) & (t < 2502.54614894971)
    valv = 14.625 * jnp.where(cond, 0.01 * jnp.sin(0.001571 * (-500.0 + tv)), 0.0)
    ridx = jax.lax.broadcasted_iota(jnp.int32, (_HEAD, 128), 0)
    cidx = jax.lax.broadcasted_iota(jnp.int32, (_HEAD, 128), 1)
    first = (ridx == 0) & (cidx == 0)
    vbuf[...] = jnp.where(first, valv, vbuf[...])

    head_out = pltpu.make_async_copy(vbuf, o_ref.at[pl.ds(0, _HEAD), :], hsem)
    head_out.start()
    head_out.wait()
    for cp in bulk:
        cp.wait()


def kernel(y, w, c, t):
    w2 = w.reshape(_ROWS, 128)
    t1 = t.reshape(1)
    out = pl.pallas_call(
        _body,
        in_specs=[
            pl.BlockSpec(memory_space=pltpu.SMEM),
            pl.BlockSpec(memory_space=pl.ANY),
        ],
        out_specs=pl.BlockSpec(memory_space=pl.ANY),
        out_shape=jax.ShapeDtypeStruct((_ROWS, 128), jnp.float32),
        scratch_shapes=[pltpu.VMEM((_HEAD, 128), jnp.float32),
                        pltpu.SemaphoreType.DMA]
                       + [pltpu.SemaphoreType.DMA] * _NBULK,
    )(t1, w2)
    return out.reshape(_N)


# speedup vs baseline: 43.2749x; 43.2749x over previous
import jax
import jax.numpy as jnp
from jax.experimental import pallas as pl
from jax.experimental.pallas import tpu as pltpu

_N = 8388608
_ROWS = 65536
_GRID = 8
_BLOCK_ROWS = _ROWS // _GRID


def _body(t_ref, w_ref, o_ref):
    o_ref[...] = w_ref[...]

    @pl.when(pl.program_id(0) == 0)
    def _():
        t = t_ref[0]
        tv = jnp.full((8, 128), t, dtype=jnp.float32)
        cond = (t > 500.0) & (t < 2502.54614894971)
        valv = 14.625 * jnp.where(cond, 0.01 * jnp.sin(0.001571 * (-500.0 + tv)), 0.0)
        ridx = jax.lax.broadcasted_iota(jnp.int32, (8, 128), 0)
        cidx = jax.lax.broadcasted_iota(jnp.int32, (8, 128), 1)
        first = (ridx == 0) & (cidx == 0)
        o_ref[0:8, :] = jnp.where(first, valv, w_ref[0:8, :])


def kernel(y, w, c, t):
    w2 = w.reshape(_ROWS, 128)
    t1 = t.reshape(1)
    out = pl.pallas_call(
        _body,
        grid=(_GRID,),
        in_specs=[
            pl.BlockSpec(memory_space=pltpu.SMEM),
            pl.BlockSpec((_BLOCK_ROWS, 128), lambda i: (i, 0)),
        ],
        out_specs=pl.BlockSpec((_BLOCK_ROWS, 128), lambda i: (i, 0)),
        out_shape=jax.ShapeDtypeStruct((_ROWS, 128), jnp.float32),
    )(t1, w2)
    return out.reshape(_N)


# speedup vs baseline: 46.2496x; 1.0687x over previous
import jax
import jax.numpy as jnp
from jax.experimental import pallas as pl
from jax.experimental.pallas import tpu as pltpu

_N = 8388608
_ROWS = 65536
_GRID = 4
_BLOCK_ROWS = _ROWS // _GRID


def _body(t_ref, w_ref, o_ref):
    o_ref[...] = w_ref[...]

    @pl.when(pl.program_id(0) == 0)
    def _():
        t = t_ref[0]
        tv = jnp.full((8, 128), t, dtype=jnp.float32)
        cond = (t > 500.0) & (t < 2502.54614894971)
        valv = 14.625 * jnp.where(cond, 0.01 * jnp.sin(0.001571 * (-500.0 + tv)), 0.0)
        ridx = jax.lax.broadcasted_iota(jnp.int32, (8, 128), 0)
        cidx = jax.lax.broadcasted_iota(jnp.int32, (8, 128), 1)
        first = (ridx == 0) & (cidx == 0)
        o_ref[0:8, :] = jnp.where(first, valv, w_ref[0:8, :])


def kernel(y, w, c, t):
    w2 = w.reshape(_ROWS, 128)
    t1 = t.reshape(1)
    out = pl.pallas_call(
        _body,
        grid=(_GRID,),
        in_specs=[
            pl.BlockSpec(memory_space=pltpu.SMEM),
            pl.BlockSpec((_BLOCK_ROWS, 128), lambda i: (i, 0)),
        ],
        out_specs=pl.BlockSpec((_BLOCK_ROWS, 128), lambda i: (i, 0)),
        out_shape=jax.ShapeDtypeStruct((_ROWS, 128), jnp.float32),
    )(t1, w2)
    return out.reshape(_N)
